# speedup vs baseline: 2.0697x; 2.0697x over previous
import functools

import jax
import jax.numpy as jnp
from jax import lax
from jax.experimental import pallas as pl
from jax.experimental.pallas import tpu as pltpu
from jax.experimental.pallas import tpu_sc as plsc

EF = 128
KNN = 8

_NC = 2
_NS = 16
_NW = _NC * _NS

_CH = 840
_CPW = 15


def _h_matmul_body(x_ref, w_ref, o_ref):
    o_ref[...] = jnp.dot(x_ref[...], w_ref[...],
                         preferred_element_type=jnp.float32)


def _h_matmul(x, w_t):
    n = x.shape[0]
    bm = 2000
    grid = n // bm
    return pl.pallas_call(
        _h_matmul_body,
        grid=(grid,),
        in_specs=[
            pl.BlockSpec((bm, EF), lambda i: (i, 0)),
            pl.BlockSpec((EF, EF), lambda i: (0, 0)),
        ],
        out_specs=pl.BlockSpec((bm, EF), lambda i: (i, 0)),
        out_shape=jax.ShapeDtypeStruct((n, EF), jnp.float32),
    )(x, w_t)


def _sc_gather_body(h_hbm, idx_hbm, out_hbm, idx_v, rows_v, sem):
    wid = lax.axis_index("s") * _NC + lax.axis_index("c")
    base = wid * (_CH * _CPW)

    def body(i, carry):
        off = base + i * _CH
        pltpu.sync_copy(idx_hbm.at[pl.ds(off, _CH)], idx_v)
        pltpu.async_copy(h_hbm.at[idx_v], rows_v, sem).wait()
        pltpu.sync_copy(rows_v, out_hbm.at[pl.ds(off, _CH)])
        return carry

    lax.fori_loop(0, _CPW, body, 0)


def _sc_gather(h, idx_pad, e_pad):
    mesh = plsc.VectorSubcoreMesh(core_axis_name="c", subcore_axis_name="s")
    k = pl.kernel(
        _sc_gather_body,
        out_type=jax.ShapeDtypeStruct((e_pad, EF), jnp.float32),
        mesh=mesh,
        scratch_types=[
            pltpu.VMEM((_CH,), jnp.int32),
            pltpu.VMEM((_CH, EF), jnp.float32),
            pltpu.SemaphoreType.DMA,
        ],
    )
    return k(h, idx_pad)


def _mlp_max_body(g_ref, xyz_ref, w1x_ref, b1_ref, w2t_ref, b2_ref, o_ref):
    pre = g_ref[...] + b1_ref[...]
    for j in range(3):
        pre += xyz_ref[0, j, :][:, None] * w1x_ref[j, :][None, :]
    act = jnp.where(pre >= 0, pre, 0.01 * pre)
    o2 = jnp.dot(act, w2t_ref[...],
                 preferred_element_type=jnp.float32) + b2_ref[...]
    bm = o2.shape[0]
    o_ref[...] = jnp.max(o2.reshape(bm // KNN, KNN, EF), axis=1)


def _mlp_max(g, xyz_t, w1x_t, b1, w2_t, b2, n_nodes):
    e = n_nodes * KNN
    bm = 3200
    grid = e // bm
    return pl.pallas_call(
        _mlp_max_body,
        grid=(grid,),
        in_specs=[
            pl.BlockSpec((bm, EF), lambda i: (i, 0)),
            pl.BlockSpec((1, 3, bm), lambda i: (i, 0, 0)),
            pl.BlockSpec((8, EF), lambda i: (0, 0)),
            pl.BlockSpec((1, EF), lambda i: (0, 0)),
            pl.BlockSpec((EF, EF), lambda i: (0, 0)),
            pl.BlockSpec((1, EF), lambda i: (0, 0)),
        ],
        out_specs=pl.BlockSpec((bm // KNN, EF), lambda i: (i, 0)),
        out_shape=jax.ShapeDtypeStruct((n_nodes, EF), jnp.float32),
    )(g, xyz_t, w1x_t, b1, w2_t, b2)


def kernel(input, KNN_idx, KNN_xyz, W1, b1, W2, b2):
    n = input.shape[0]
    e = KNN_idx.shape[0]

    idx = KNN_idx.astype(jnp.int32)
    e_pad = _NW * _CPW * _CH
    idx_pad = jnp.concatenate(
        [idx, jnp.zeros((e_pad - e,), dtype=jnp.int32)])

    w1f_t = W1[:, :EF].T
    w1x_t = jnp.zeros((8, EF), jnp.float32).at[:3].set(W1[:, EF:].T)
    w2_t = W2.T

    h = _h_matmul(input, w1f_t)
    g = _sc_gather(h, idx_pad, e_pad)

    bm = 3200
    xyz_t = KNN_xyz.reshape(e // bm, bm, 3).transpose(0, 2, 1)

    return _mlp_max(g, xyz_t, w1x_t, b1.reshape(1, EF), w2_t,
                    b2.reshape(1, EF), n)


# speedup vs baseline: 2.5006x; 1.2082x over previous
import functools

import jax
import jax.numpy as jnp
from jax import lax
from jax.experimental import pallas as pl
from jax.experimental.pallas import tpu as pltpu
from jax.experimental.pallas import tpu_sc as plsc

EF = 128
KNN = 8

_NC = 2
_NS = 16
_NW = _NC * _NS

_CH = 448
_CPW = 28


def _h_matmul_body(x_ref, w_ref, o_ref):
    o_ref[...] = jnp.dot(x_ref[...], w_ref[...],
                         preferred_element_type=jnp.float32)


def _h_matmul(x, w_t):
    n = x.shape[0]
    bm = 2000
    grid = n // bm
    return pl.pallas_call(
        _h_matmul_body,
        grid=(grid,),
        in_specs=[
            pl.BlockSpec((bm, EF), lambda i: (i, 0)),
            pl.BlockSpec((EF, EF), lambda i: (0, 0)),
        ],
        out_specs=pl.BlockSpec((bm, EF), lambda i: (i, 0)),
        out_shape=jax.ShapeDtypeStruct((n, EF), jnp.float32),
    )(x, w_t)


def _sc_gather_body(h_hbm, idx_hbm, out_hbm, idx_v0, idx_v1, rows_v,
                    gsem, ssem):
    wid = lax.axis_index("s") * _NC + lax.axis_index("c")
    base = wid * (_CH * _CPW)
    idx_v = [idx_v0, idx_v1]

    store = [None, None]
    gath = [None, None]

    def start(i):
        b = i % 2
        pltpu.sync_copy(idx_hbm.at[pl.ds(base + i * _CH, _CH)], idx_v[b])
        gath[b] = pltpu.async_copy(h_hbm.at[idx_v[b]], rows_v.at[b],
                                   gsem.at[b])

    start(0)
    for i in range(_CPW):
        b = i % 2
        if i + 1 < _CPW:
            if store[1 - b] is not None:
                store[1 - b].wait()
            start(i + 1)
        gath[b].wait()
        store[b] = pltpu.async_copy(
            rows_v.at[b], out_hbm.at[pl.ds(base + i * _CH, _CH)], ssem.at[b])
    store[0].wait()
    store[1].wait()


def _sc_gather(h, idx_pad, e_pad):
    mesh = plsc.VectorSubcoreMesh(core_axis_name="c", subcore_axis_name="s")
    k = pl.kernel(
        _sc_gather_body,
        out_type=jax.ShapeDtypeStruct((e_pad, EF), jnp.float32),
        mesh=mesh,
        scratch_types=[
            pltpu.VMEM((_CH,), jnp.int32),
            pltpu.VMEM((_CH,), jnp.int32),
            pltpu.VMEM((2, _CH, EF), jnp.float32),
            pltpu.SemaphoreType.DMA((2,)),
            pltpu.SemaphoreType.DMA((2,)),
        ],
    )
    return k(h, idx_pad)


def _mlp_max_body(g_ref, xyz_ref, w1x_ref, b1_ref, w2t_ref, b2_ref, o_ref):
    pre = g_ref[...] + b1_ref[...]
    for j in range(3):
        pre += xyz_ref[0, j, :][:, None] * w1x_ref[j, :][None, :]
    act = jnp.where(pre >= 0, pre, 0.01 * pre)
    o2 = jnp.dot(act.astype(jnp.bfloat16), w2t_ref[...],
                 preferred_element_type=jnp.float32) + b2_ref[...]
    bm = o2.shape[0]
    o_ref[...] = jnp.max(o2.reshape(bm // KNN, KNN, EF), axis=1)


def _mlp_max(g, xyz_t, w1x_t, b1, w2_t, b2, n_nodes):
    e = n_nodes * KNN
    bm = 3200
    grid = e // bm
    return pl.pallas_call(
        _mlp_max_body,
        grid=(grid,),
        in_specs=[
            pl.BlockSpec((bm, EF), lambda i: (i, 0)),
            pl.BlockSpec((1, 3, bm), lambda i: (i, 0, 0)),
            pl.BlockSpec((8, EF), lambda i: (0, 0)),
            pl.BlockSpec((1, EF), lambda i: (0, 0)),
            pl.BlockSpec((EF, EF), lambda i: (0, 0)),
            pl.BlockSpec((1, EF), lambda i: (0, 0)),
        ],
        out_specs=pl.BlockSpec((bm // KNN, EF), lambda i: (i, 0)),
        out_shape=jax.ShapeDtypeStruct((n_nodes, EF), jnp.float32),
    )(g, xyz_t, w1x_t, b1, w2_t, b2)


def kernel(input, KNN_idx, KNN_xyz, W1, b1, W2, b2):
    n = input.shape[0]
    e = KNN_idx.shape[0]

    idx = KNN_idx.astype(jnp.int32)
    e_pad = _NW * _CPW * _CH
    idx_pad = jnp.concatenate(
        [idx, jnp.zeros((e_pad - e,), dtype=jnp.int32)])

    w1f_t = W1[:, :EF].T
    w1x_t = jnp.zeros((8, EF), jnp.float32).at[:3].set(W1[:, EF:].T)
    w2_t = W2.T.astype(jnp.bfloat16)

    h = _h_matmul(input, w1f_t)
    g = _sc_gather(h, idx_pad, e_pad)

    bm = 3200
    xyz_t = KNN_xyz.reshape(e // bm, bm, 3).transpose(0, 2, 1)

    return _mlp_max(g, xyz_t, w1x_t, b1.reshape(1, EF), w2_t,
                    b2.reshape(1, EF), n)
